# baseline (speedup 1.0000x reference)
import functools

import jax
import jax.numpy as jnp
from jax import lax
from jax.experimental import pallas as pl
from jax.experimental.pallas import tpu as pltpu
from jax.experimental.pallas import tpu_sc as plsc

ROWS = 1024
COLS = 100000
KTOP = 5

LANES = 16
NCORES = 2
NSUB = 16
NWORKERS = NCORES * NSUB
RPW = ROWS // NWORKERS
RPS = ROWS // NCORES
GW = 128
CMAX = COLS - 160
TCUT = CMAX + GW
TAILBLK = TCUT // 128

SCCOLS = 24576
SLABW = 1536
NSLAB = SCCOLS // SLABW
CPS = SLABW // LANES

CBLK = 4096
KCH = CBLK // 128
B0 = SCCOLS // CBLK
NTC = -(-(COLS - SCCOLS) // CBLK)
LASTB = B0 + NTC - 1
TAILV = COLS - LASTB * CBLK
TAILK = TAILV // 128
TAILR = TAILV - TAILK * 128


def _sc_gather_body(logits2d, targets_hbm, chunks_out, tgt_s, sem):
    cid = lax.axis_index("c")
    base = cid * RPS
    pltpu.sync_copy(targets_hbm.at[pl.ds(base, RPS)], tgt_s)

    def issue(i, _):
        t_s = tgt_s[i]
        col0 = pl.multiple_of(
            jnp.minimum(lax.bitwise_and(t_s, -GW), CMAX), GW)
        row = base + i
        r0 = pl.multiple_of(lax.bitwise_and(row, -8), 8)
        pltpu.async_copy(
            logits2d.at[pl.ds(r0, 8), pl.ds(col0, GW)],
            chunks_out.at[row], sem)
        return 0

    lax.fori_loop(0, RPS, issue, 0, unroll=8)

    def drain(i, _):
        pltpu.make_async_copy(
            logits2d.at[pl.ds(0, 8), pl.ds(0, GW)],
            chunks_out.at[0], sem).wait()
        return 0

    lax.fori_loop(0, RPS, drain, 0, unroll=8)


@functools.lru_cache(maxsize=1)
def _sc_gather():
    mesh = plsc.ScalarSubcoreMesh(axis_name="c", num_cores=NCORES)
    return pl.kernel(
        _sc_gather_body,
        out_type=jax.ShapeDtypeStruct((ROWS, 8, GW), jnp.float32),
        mesh=mesh,
        scratch_types=[
            pltpu.SMEM((RPS,), jnp.int32),
            pltpu.SemaphoreType.DMA,
        ],
    )


def _extract_body(chunks_ref, tailblk_ref, tgt_ref, thr128_ref, thi128_ref,
                  tgt128_ref, thr16_ref, thi16_ref, tgt16_ref):
    tgt = tgt_ref[...]
    col0 = jnp.minimum(tgt & -GW, CMAX)
    lane3 = (tgt - col0).reshape(ROWS, 1, 1)
    r3 = lax.broadcasted_iota(jnp.int32, (ROWS, 8, GW), 0)
    s3 = lax.broadcasted_iota(jnp.int32, (ROWS, 8, GW), 1)
    l3 = lax.broadcasted_iota(jnp.int32, (ROWS, 8, GW), 2)
    oh3 = (s3 == (r3 & 7)) & (l3 == lane3)
    t_chunk = jnp.sum(
        jnp.sum(jnp.where(oh3, chunks_ref[...], 0.0), axis=2),
        axis=1, keepdims=True)
    colv = lax.broadcasted_iota(jnp.int32, (ROWS, 128), 1)
    oh = colv == (tgt - TAILBLK * 128)
    hit = jnp.sum(jnp.where(oh, tailblk_ref[...], 0.0), axis=1, keepdims=True)
    t = jnp.where(tgt >= TCUT, hit, t_chunk)
    bb = lax.bitcast_convert_type(t + 0.0, jnp.int32)
    thi = lax.bitcast_convert_type(
        jnp.where(bb >= 0, bb + 1, bb - 1), jnp.float32)
    thr128_ref[...] = jnp.broadcast_to(t, (ROWS, 128))
    thi128_ref[...] = jnp.broadcast_to(thi, (ROWS, 128))
    tgt128_ref[...] = jnp.broadcast_to(tgt, (ROWS, 128))
    thr16_ref[...] = jnp.broadcast_to(t, (ROWS, LANES))
    thi16_ref[...] = jnp.broadcast_to(thi, (ROWS, LANES))
    tgt16_ref[...] = jnp.broadcast_to(tgt, (ROWS, LANES))


def _extract(chunks, logits, tgt2):
    f32 = jnp.float32
    return pl.pallas_call(
        _extract_body,
        grid=(1,),
        in_specs=[
            pl.BlockSpec((ROWS, 8, GW), lambda c: (0, 0, 0)),
            pl.BlockSpec((ROWS, 128), lambda c: (0, TAILBLK)),
            pl.BlockSpec((ROWS, 1), lambda c: (0, 0)),
        ],
        out_specs=[pl.BlockSpec((ROWS, 128), lambda c: (0, 0))] * 3
        + [pl.BlockSpec((ROWS, LANES), lambda c: (0, 0))] * 3,
        out_shape=[jax.ShapeDtypeStruct((ROWS, 128), f32),
                   jax.ShapeDtypeStruct((ROWS, 128), f32),
                   jax.ShapeDtypeStruct((ROWS, 128), jnp.int32),
                   jax.ShapeDtypeStruct((ROWS, LANES), f32),
                   jax.ShapeDtypeStruct((ROWS, LANES), f32),
                   jax.ShapeDtypeStruct((ROWS, LANES), jnp.int32)],
    )(chunks, logits, tgt2)


UCH = 4


def _sc_count_body(logits2d, thr16_h, thi16_h, tgt16_h, out_h,
                   thr_v, thi_v, tgt_v, buf0, buf1, acc_v, col_v, sem0, sem1):
    wid = lax.axis_index("s") * NCORES + lax.axis_index("c")
    base = wid * RPW
    pltpu.sync_copy(thr16_h.at[pl.ds(base, RPW)], thr_v)
    pltpu.sync_copy(thi16_h.at[pl.ds(base, RPW)], thi_v)
    pltpu.sync_copy(tgt16_h.at[pl.ds(base, RPW)], tgt_v)
    for r in range(RPW):
        acc_v[r, :] = jnp.zeros((LANES,), jnp.int32)
    bufs = [buf0, buf1]
    sems = [sem0, sem1]
    iota = lax.iota(jnp.int32, LANES)
    cps = {0: pltpu.async_copy(
        logits2d.at[pl.ds(base, RPW), pl.ds(0, SLABW)], buf0, sem0)}
    for g in range(NSLAB):
        if g + 1 < NSLAB:
            cps[g + 1] = pltpu.async_copy(
                logits2d.at[pl.ds(base, RPW), pl.ds((g + 1) * SLABW, SLABW)],
                bufs[(g + 1) % 2], sems[(g + 1) % 2])
        cps[g].wait()
        buf = bufs[g % 2]
        col_v[...] = iota + (g * SLABW)

        def step(jb, _):
            cv = col_v[...]
            for r in range(RPW):
                thr = thr_v[r, :]
                thi = thi_v[r, :]
                tg = tgt_v[r, :]
                cnt = None
                for jj in range(UCH):
                    off = pl.multiple_of(jb * (UCH * LANES) + jj * LANES,
                                         LANES)
                    v = buf[r, pl.ds(off, LANES)]
                    mlt = (cv + jj * LANES) < tg
                    m = v >= jnp.where(mlt, thr, thi)
                    c1 = jnp.where(m, 1, 0)
                    cnt = c1 if cnt is None else cnt + c1
                acc_v[r, :] = acc_v[r, :] + cnt
            col_v[...] = cv + (UCH * LANES)
            return 0

        lax.fori_loop(0, CPS // UCH, step, 0)
    pltpu.sync_copy(acc_v, out_h.at[pl.ds(base, RPW)])


@functools.lru_cache(maxsize=1)
def _sc_count():
    mesh = plsc.VectorSubcoreMesh(core_axis_name="c", subcore_axis_name="s",
                                  num_cores=NCORES, num_subcores=NSUB)
    return pl.kernel(
        _sc_count_body,
        out_type=jax.ShapeDtypeStruct((ROWS, LANES), jnp.int32),
        mesh=mesh,
        scratch_types=[
            pltpu.VMEM((RPW, LANES), jnp.float32),
            pltpu.VMEM((RPW, LANES), jnp.float32),
            pltpu.VMEM((RPW, LANES), jnp.int32),
            pltpu.VMEM((RPW, SLABW), jnp.float32),
            pltpu.VMEM((RPW, SLABW), jnp.float32),
            pltpu.VMEM((RPW, LANES), jnp.int32),
            pltpu.VMEM((LANES,), jnp.int32),
            pltpu.SemaphoreType.DMA,
            pltpu.SemaphoreType.DMA,
        ],
    )


def _tc_body(logits_ref, thr_ref, thi_ref, tgt_ref, out_ref, acc_ref):
    c = pl.program_id(0)
    b = c + B0

    colv = lax.broadcasted_iota(jnp.int32, (ROWS, 128), 1)

    def slice_count(k, extra_mask=None):
        vk = logits_ref[:, k * 128:(k + 1) * 128]
        mlt = colv < (tgt_ref[...] - (b * CBLK + k * 128))
        m = vk >= jnp.where(mlt, thr_ref[...], thi_ref[...])
        if extra_mask is not None:
            m = m & extra_mask
        return m.astype(jnp.int32)

    @pl.when(c == 0)
    def _init():
        acc_ref[...] = jnp.zeros_like(acc_ref)

    @pl.when(c < NTC - 1)
    def _main():
        s = slice_count(0)
        for k in range(1, KCH):
            s += slice_count(k)
        acc_ref[...] += s

    @pl.when(c == NTC - 1)
    def _last():
        s = slice_count(0)
        for k in range(1, TAILK):
            s += slice_count(k)
        s += slice_count(TAILK, extra_mask=colv < TAILR)
        out_ref[...] = acc_ref[...] + s


def _tc_count(logits, thr128, thi128, tgt128):
    return pl.pallas_call(
        _tc_body,
        grid=(NTC,),
        in_specs=[
            pl.BlockSpec((ROWS, CBLK), lambda c: (0, c + B0)),
            pl.BlockSpec((ROWS, 128), lambda c: (0, 0)),
            pl.BlockSpec((ROWS, 128), lambda c: (0, 0)),
            pl.BlockSpec((ROWS, 128), lambda c: (0, 0)),
        ],
        out_specs=pl.BlockSpec((ROWS, 128), lambda c: (0, 0)),
        out_shape=jax.ShapeDtypeStruct((ROWS, 128), jnp.int32),
        scratch_shapes=[pltpu.VMEM((ROWS, 128), jnp.int32)],
    )(logits, thr128, thi128, tgt128)


def _combine_body(acc_ref, scp_ref, pm_ref, out_ref):
    cnt = (jnp.sum(acc_ref[...], axis=1, keepdims=True)
           + jnp.sum(scp_ref[...], axis=1, keepdims=True))
    pm = pm_ref[...]
    correct = jnp.where(cnt < KTOP, pm, 0.0)
    out_ref[0, 0] = jnp.sum(correct) / jnp.sum(pm)


def _combine(acc, scp, pm2):
    return pl.pallas_call(
        _combine_body,
        grid=(1,),
        in_specs=[
            pl.BlockSpec((ROWS, 128), lambda c: (0, 0)),
            pl.BlockSpec((ROWS, LANES), lambda c: (0, 0)),
            pl.BlockSpec((ROWS, 1), lambda c: (0, 0)),
        ],
        out_specs=pl.BlockSpec(memory_space=pltpu.SMEM),
        out_shape=jax.ShapeDtypeStruct((1, 1), jnp.float32),
    )(acc, scp, pm2)


def kernel(logits, targets, padding_mask):
    tgt = targets.astype(jnp.int32)
    tgt2 = tgt.reshape(ROWS, 1)
    chunks = _sc_gather()(logits, tgt)
    thr128, thi128, tgt128, thr16, thi16, tgt16 = _extract(
        chunks, logits, tgt2)
    scp = _sc_count()(logits, thr16, thi16, tgt16)
    acc = _tc_count(logits, thr128, thi128, tgt128)
    pm2 = padding_mask.astype(jnp.float32).reshape(ROWS, 1)
    return _combine(acc, scp, pm2)[0, 0]


# speedup vs baseline: 1.0212x; 1.0212x over previous
import functools

import jax
import jax.numpy as jnp
from jax import lax
from jax.experimental import pallas as pl
from jax.experimental.pallas import tpu as pltpu
from jax.experimental.pallas import tpu_sc as plsc

ROWS = 1024
COLS = 100000
KTOP = 5

LANES = 16
NCORES = 2
NSUB = 16
NWORKERS = NCORES * NSUB
RPW = ROWS // NWORKERS
GW = 128
CMAX = COLS - 160

CBLK = 4096
KCH = CBLK // 128
NBLK = -(-COLS // CBLK)
LASTB = NBLK - 1
TAILV = COLS - LASTB * CBLK
TAILK = TAILV // 128
TAILR = TAILV - TAILK * 128
TSTART = LASTB * CBLK
TCUT = CMAX + GW


RPS = ROWS // NCORES


def _sc_gather_body(logits2d, targets_hbm, chunks_out, tgt_s, sem):
    cid = lax.axis_index("c")
    base = cid * RPS
    pltpu.sync_copy(targets_hbm.at[pl.ds(base, RPS)], tgt_s)

    def issue(i, _):
        t_s = tgt_s[i]
        col0 = pl.multiple_of(
            jnp.minimum(lax.bitwise_and(t_s, -GW), CMAX), GW)
        row = base + i
        r0 = pl.multiple_of(lax.bitwise_and(row, -8), 8)
        pltpu.async_copy(
            logits2d.at[pl.ds(r0, 8), pl.ds(col0, GW)],
            chunks_out.at[row], sem)
        return 0

    lax.fori_loop(0, RPS, issue, 0, unroll=8)

    def drain(i, _):
        pltpu.make_async_copy(
            logits2d.at[pl.ds(0, 8), pl.ds(0, GW)],
            chunks_out.at[0], sem).wait()
        return 0

    lax.fori_loop(0, RPS, drain, 0, unroll=8)


@functools.lru_cache(maxsize=1)
def _sc_gather():
    mesh = plsc.ScalarSubcoreMesh(axis_name="c", num_cores=NCORES)
    return pl.kernel(
        _sc_gather_body,
        out_type=jax.ShapeDtypeStruct((ROWS, 8, GW), jnp.float32),
        mesh=mesh,
        scratch_types=[
            pltpu.SMEM((RPS,), jnp.int32),
            pltpu.SemaphoreType.DMA,
        ],
    )


def _tc_body(logits_ref, chunks_ref, tgt_ref, pm_ref, out_ref, acc_ref,
             thr_ref, thi_ref, tgtb_ref):
    c = pl.program_id(0)
    b = jnp.where(c == 0, LASTB, c - 1)

    colv = lax.broadcasted_iota(jnp.int32, (ROWS, 128), 1)

    def slice_count(k, extra_mask=None):
        vk = logits_ref[:, k * 128:(k + 1) * 128]
        mlt = colv < (tgtb_ref[...] - (b * CBLK + k * 128))
        m = vk >= jnp.where(mlt, thr_ref[...], thi_ref[...])
        if extra_mask is not None:
            m = m & extra_mask
        return m.astype(jnp.int32)

    @pl.when(c == 0)
    def _init():
        tgt = tgt_ref[...]
        col0 = jnp.minimum(tgt & -GW, CMAX)
        lane3 = (tgt - col0).reshape(ROWS, 1, 1)
        r3 = lax.broadcasted_iota(jnp.int32, (ROWS, 8, GW), 0)
        s3 = lax.broadcasted_iota(jnp.int32, (ROWS, 8, GW), 1)
        l3 = lax.broadcasted_iota(jnp.int32, (ROWS, 8, GW), 2)
        oh3 = (s3 == (r3 & 7)) & (l3 == lane3)
        t_chunk = jnp.sum(
            jnp.sum(jnp.where(oh3, chunks_ref[...], 0.0), axis=2),
            axis=1, keepdims=True)
        tgt_rel = tgt - TSTART
        vk = logits_ref[:, TAILK * 128:(TAILK + 1) * 128]
        oh = colv == (tgt_rel - TAILK * 128)
        hit = jnp.sum(jnp.where(oh, vk, 0.0), axis=1, keepdims=True)
        t = jnp.where(tgt >= TCUT, hit, t_chunk)
        thr_ref[...] = jnp.broadcast_to(t, (ROWS, 128))
        bb = lax.bitcast_convert_type(t + 0.0, jnp.int32)
        bhi = jnp.where(bb >= 0, bb + 1, bb - 1)
        thi_ref[...] = jnp.broadcast_to(
            lax.bitcast_convert_type(bhi, jnp.float32), (ROWS, 128))
        tgtb_ref[...] = jnp.broadcast_to(tgt, (ROWS, 128))
        s = slice_count(0)
        for k in range(1, TAILK):
            s += slice_count(k)
        s += slice_count(TAILK, extra_mask=colv < TAILR)
        acc_ref[...] = s

    @pl.when(c > 0)
    def _main():
        s = slice_count(0)
        for k in range(1, KCH):
            s += slice_count(k)
        acc_ref[...] += s

    @pl.when(c == NBLK - 1)
    def _fin():
        cnt = jnp.sum(acc_ref[...], axis=1, keepdims=True)
        pm = pm_ref[...]
        correct = jnp.where(cnt < KTOP, pm, 0.0)
        out_ref[0, 0] = jnp.sum(correct) / jnp.sum(pm)


def _tc_accuracy(logits, chunks, tgt2, pm2):
    return pl.pallas_call(
        _tc_body,
        grid=(NBLK,),
        in_specs=[
            pl.BlockSpec((ROWS, CBLK),
                         lambda c: (0, jnp.where(c == 0, LASTB, c - 1))),
            pl.BlockSpec((ROWS, 8, GW), lambda c: (0, 0, 0)),
            pl.BlockSpec((ROWS, 1), lambda c: (0, 0)),
            pl.BlockSpec((ROWS, 1), lambda c: (0, 0)),
        ],
        out_specs=pl.BlockSpec(memory_space=pltpu.SMEM),
        out_shape=jax.ShapeDtypeStruct((1, 1), jnp.float32),
        scratch_shapes=[
            pltpu.VMEM((ROWS, 128), jnp.int32),
            pltpu.VMEM((ROWS, 128), jnp.float32),
            pltpu.VMEM((ROWS, 128), jnp.float32),
            pltpu.VMEM((ROWS, 128), jnp.int32),
        ],
    )(logits, chunks, tgt2, pm2)


def kernel(logits, targets, padding_mask):
    tgt = targets.astype(jnp.int32)
    chunks = _sc_gather()(logits, tgt)
    acc = _tc_accuracy(
        logits,
        chunks,
        tgt.reshape(ROWS, 1),
        padding_mask.astype(jnp.float32).reshape(ROWS, 1),
    )
    return acc[0, 0]
